# baseline (speedup 1.0000x reference)
import jax
import jax.numpy as jnp
import numpy as np
from jax.experimental import pallas as pl

VOX = 48
WIDTH = 128
EMB = 512
KF = 8
CAP = 64
STAGES = [(2048, 2.5), (512, 5.0), (128, 9.0)]

_BANDS = ((2.0 ** np.arange(KF, dtype=np.float32)) * np.pi).reshape(1, KF)


def _cell_pad(target_k):
    cell = max(1, int(np.ceil(VOX / target_k ** (1.0 / 3.0))))
    pad = ((VOX - 1) // cell + 1) ** 3
    return cell, pad


def _subsample_idx(xyz_int, src_valid, target_k, pad):
    cell, _ = _cell_pad(target_k)
    cells = xyz_int // cell
    h = cells[:, 0] + cells[:, 1] * 73856093 + cells[:, 2] * 19349663
    sentinel = jnp.iinfo(h.dtype).max
    h = jnp.where(src_valid, h, sentinel)
    order = jnp.argsort(h)
    hs = h[order]
    first = jnp.concatenate([jnp.ones((1,), dtype=bool), hs[1:] != hs[:-1]])
    keep_flag = first & src_valid[order]
    cnt = jnp.cumsum(keep_flag)
    dest = jnp.where(keep_flag, cnt - 1, pad)
    keep = jnp.zeros((pad,), dtype=order.dtype).at[dest].set(order, mode='drop')
    kvalid = jnp.arange(pad) < cnt[-1]
    return keep, kvalid


def _geometry(coords):
    xyz = coords.astype(jnp.float32)
    src = xyz
    src_valid = jnp.ones((xyz.shape[0],), dtype=bool)
    geo = []
    for (K, r) in STAGES:
        _, pad = _cell_pad(K)
        keep, c_valid = _subsample_idx(src.astype(jnp.int32), src_valid, K, pad)
        centers = src[keep]
        d2 = ((centers[:, None, :] - src[None, :, :]) ** 2).sum(-1)
        d2 = jnp.maximum(d2, 0.0)
        d2 = jnp.where(src_valid[None, :], d2, jnp.inf)
        d2m = jnp.where(d2 > r * r, jnp.inf, d2)
        k_take = min(CAP, src.shape[0])
        order = jnp.argsort(d2m, axis=1)[:, :k_take]
        vals = jnp.take_along_axis(d2m, order, axis=1)
        none = jnp.isinf(vals[:, 0])
        fb = d2.argmin(1)
        fb_vals = jnp.take_along_axis(d2, fb[:, None], axis=1)[:, 0]
        order = order.at[:, 0].set(jnp.where(none, fb, order[:, 0]))
        vals = vals.at[:, 0].set(jnp.where(none, fb_vals, vals[:, 0]))
        valid = ~jnp.isinf(vals)
        vals = jnp.where(valid, vals, 0.0)
        nbr_xyz = src[order]
        rr = max(r, 1e-06)
        delta = (nbr_xyz - centers[:, None, :]) / rr
        distn = jnp.sqrt(vals)[..., None] / rr
        geo.append((order, delta.astype(jnp.float32), distn.astype(jnp.float32),
                    valid))
        src = centers
        src_valid = c_valid
    return geo, src_valid


def _mlp_body(feat, vm, w1, b1, w2, b2, wp, bp, bc):
    bands = jnp.exp2(jax.lax.broadcasted_iota(
        jnp.int32, (1, KF), 1).astype(jnp.float32)) * np.pi
    parts = [feat]
    for d in range(3):
        ang = feat[:, d:d + 1] * bands
        parts.append(jnp.sin(ang))
        parts.append(jnp.cos(ang))
    x = jnp.concatenate(parts, axis=1)
    h = jnp.maximum(jnp.dot(x, w1, preferred_element_type=jnp.float32) + b1, 0.0)
    h = jnp.maximum(jnp.dot(h, w2, preferred_element_type=jnp.float32) + b2, 0.0)
    h = h.reshape(bc, CAP, WIDTH)
    vm3 = vm.reshape(bc, CAP, 1)
    cntv = jnp.maximum(jnp.sum(vm3, axis=1), 1.0)
    mean = jnp.sum(h * vm3, axis=1) / cntv
    mx = jnp.max(jnp.where(vm3 > 0.0, h, jnp.finfo(jnp.float32).min), axis=1)
    out = jnp.concatenate([mean, mx], axis=1)
    return jnp.maximum(jnp.dot(out, wp, preferred_element_type=jnp.float32) + bp,
                       0.0)


def _stage_kernel(feat_ref, vm_ref, w1_ref, b1_ref, w2_ref, b2_ref, wp_ref,
                  bp_ref, o_ref):
    bc = vm_ref.shape[0]
    o_ref[...] = _mlp_body(feat_ref[...], vm_ref[...], w1_ref[...], b1_ref[...],
                           w2_ref[...], b2_ref[...], wp_ref[...], bp_ref[...],
                           bc)


def _stage2_head_kernel(feat_ref, vm_ref, fm_ref, w1_ref, b1_ref, w2_ref,
                        b2_ref, wp_ref, bp_ref, wh_ref, bh_ref, o_ref):
    bc = vm_ref.shape[0]
    src2 = _mlp_body(feat_ref[...], vm_ref[...], w1_ref[...], b1_ref[...],
                     w2_ref[...], b2_ref[...], wp_ref[...], bp_ref[...], bc)
    fm = fm_ref[...]
    fcount = jnp.maximum(jnp.sum(fm, axis=0, keepdims=True), 1.0)
    gmean = jnp.sum(src2 * fm, axis=0, keepdims=True) / fcount
    gmax = jnp.max(jnp.where(fm > 0.0, src2, jnp.finfo(jnp.float32).min),
                   axis=0, keepdims=True)
    g = jnp.concatenate([gmean, gmax], axis=1)
    res = jnp.dot(g, wh_ref[...], preferred_element_type=jnp.float32) + bh_ref[...]
    o_ref[...] = jnp.broadcast_to(res, (8, EMB))


def _full_spec(a):
    return pl.BlockSpec(a.shape, lambda i: (0,) * a.ndim)


def _run_stage(feat, vm, w1, b1, w2, b2, wp, bp, bc):
    c = vm.shape[0]
    grid = (c // bc,)
    nf = feat.shape[1]
    return pl.pallas_call(
        _stage_kernel,
        grid=grid,
        in_specs=[
            pl.BlockSpec((bc * CAP, nf), lambda i: (i, 0)),
            pl.BlockSpec((bc, CAP), lambda i: (i, 0)),
            _full_spec(w1), _full_spec(b1), _full_spec(w2), _full_spec(b2),
            _full_spec(wp), _full_spec(bp),
        ],
        out_specs=pl.BlockSpec((bc, WIDTH), lambda i: (i, 0)),
        out_shape=jax.ShapeDtypeStruct((c, WIDTH), jnp.float32),
    )(feat, vm, w1, b1, w2, b2, wp, bp)


def _run_stage2_head(feat, vm, fm, w1, b1, w2, b2, wp, bp, wh, bh):
    c = vm.shape[0]
    nf = feat.shape[1]
    out = pl.pallas_call(
        _stage2_head_kernel,
        grid=(1,),
        in_specs=[
            pl.BlockSpec((c * CAP, nf), lambda i: (0, 0)),
            pl.BlockSpec((c, CAP), lambda i: (0, 0)),
            pl.BlockSpec((c, WIDTH), lambda i: (0, 0)),
            _full_spec(w1), _full_spec(b1), _full_spec(w2), _full_spec(b2),
            _full_spec(wp), _full_spec(bp), _full_spec(wh), _full_spec(bh),
        ],
        out_specs=pl.BlockSpec((8, EMB), lambda i: (0, 0)),
        out_shape=jax.ShapeDtypeStruct((8, EMB), jnp.float32),
    )(feat, vm, fm, w1, b1, w2, b2, wp, bp, wh, bh)
    return out[0]


def kernel(occupied_coords, values, bounds, drone_pos, W1_0, b1_0, W2_0, b2_0,
           Wp_0, bp_0, W1_1, b1_1, W2_1, b2_1, Wp_1, bp_1, W1_2, b1_2, W2_2,
           b2_2, Wp_2, bp_2, Wh, bh):
    geo, final_valid = _geometry(occupied_coords)
    params = [W1_0, b1_0, W2_0, b2_0, Wp_0, bp_0,
              W1_1, b1_1, W2_1, b2_1, Wp_1, bp_1,
              W1_2, b1_2, W2_2, b2_2, Wp_2, bp_2]
    src_feat = values.reshape(-1, 1)
    out = None
    for si, (nn_idx, delta, distn, valid) in enumerate(geo):
        w1, b1, w2, b2, wp, bp = params[6 * si:6 * si + 6]
        c = nn_idx.shape[0]
        nbr_feat = src_feat[nn_idx]
        f = nbr_feat.shape[-1]
        feat = jnp.concatenate(
            [delta, distn, nbr_feat], axis=-1).reshape(c * CAP, 4 + f)
        vm = valid.astype(jnp.float32)
        b1r = b1.reshape(1, WIDTH)
        b2r = b2.reshape(1, WIDTH)
        bpr = bp.reshape(1, WIDTH)
        if si < 2:
            bc = 64
            cp = ((c + bc - 1) // bc) * bc
            npad = cp - c
            if npad:
                feat = jnp.concatenate(
                    [feat, jnp.zeros((npad * CAP, 4 + f), jnp.float32)], axis=0)
                vm = jnp.concatenate(
                    [vm, jnp.ones((npad, CAP), jnp.float32)], axis=0)
            src_feat = _run_stage(feat, vm, w1, b1r, w2, b2r, wp, bpr, bc)[:c]
        else:
            cp = 128
            npad = cp - c
            featp = jnp.concatenate(
                [feat, jnp.zeros((npad * CAP, 4 + f), jnp.float32)], axis=0)
            vmp = jnp.concatenate(
                [vm, jnp.ones((npad, CAP), jnp.float32)], axis=0)
            fmask = jnp.concatenate(
                [final_valid.astype(jnp.float32), jnp.zeros((npad,), jnp.float32)])
            fm = jnp.broadcast_to(fmask[:, None], (cp, WIDTH))
            out = _run_stage2_head(featp, vmp, fm, w1, b1r, w2, b2r, wp, bpr,
                                   Wh, bh.reshape(1, EMB))
    return out
